# Initial kernel scaffold; baseline (speedup 1.0000x reference)
import jax
import jax.numpy as jnp
from jax.experimental import pallas as pl


def kernel(edge_index, emb, W1, b1, W2, b2):
    raise NotImplementedError("write your pallas kernel here")


# speedup vs baseline: 10.7872x; 10.7872x over previous
import functools

import jax
import jax.numpy as jnp
from jax import lax
from jax.experimental import pallas as pl
from jax.experimental.pallas import tpu as pltpu
from jax.experimental.pallas import tpu_sc as plsc

N = 100000
D = 32
H = D // 2
E = 1600000

LANES = 128
K = 8
NTILES = 16
ROWS = 12544
EP = ROWS * LANES
ROWS_PER_TILE = ROWS // NTILES
ITERS = ROWS_PER_TILE // K
NACC = 100096
ACC_PER_TILE = NACC // NTILES

BM = 1024
GRID = (N + BM - 1) // BM


def _sc_segment_sum(xa, xb, src2, dst2, zeros):
  mesh = plsc.VectorSubcoreMesh(core_axis_name="c", subcore_axis_name="s")

  @functools.partial(
      pl.kernel,
      out_type=(
          jax.ShapeDtypeStruct((NACC, H), jnp.float32),
          jax.ShapeDtypeStruct((NACC, H), jnp.float32),
      ),
      mesh=mesh,
      scratch_types=[
          pltpu.VMEM((K, LANES), jnp.int32),
          pltpu.VMEM((K, LANES), jnp.int32),
          pltpu.VMEM((K * LANES, H), jnp.float32),
          pltpu.VMEM_SHARED((NACC, H), jnp.float32),
          pltpu.SemaphoreType.DMA,
          pltpu.SemaphoreType.DMA,
      ],
      compiler_params=pltpu.CompilerParams(use_tc_tiling_on_sc=False),
  )
  def k(xa_hbm, xb_hbm, src_hbm, dst_hbm, z_hbm, na_hbm, nb_hbm,
        srcv, dstv, rows, acc, gsem, ssem):
    c = lax.axis_index("c")
    s = lax.axis_index("s")

    zslc = pl.ds(s * ACC_PER_TILE, ACC_PER_TILE)
    pltpu.sync_copy(z_hbm.at[zslc], acc.at[zslc])
    plsc.subcore_barrier()

    base = s * ROWS_PER_TILE

    def chunk(t, carry):
      row0 = base + t * K
      pltpu.sync_copy(src_hbm.at[pl.ds(row0, K)], srcv)
      pltpu.sync_copy(dst_hbm.at[pl.ds(row0, K)], dstv)

      @pl.when(c == 0)
      def _():
        descs = [pltpu.async_copy(xa_hbm.at[srcv.at[j]],
                                  rows.at[pl.ds(j * LANES, LANES)], gsem)
                 for j in range(K)]
        for d in descs:
          d.wait()

      @pl.when(c == 1)
      def _():
        descs = [pltpu.async_copy(xb_hbm.at[srcv.at[j]],
                                  rows.at[pl.ds(j * LANES, LANES)], gsem)
                 for j in range(K)]
        for d in descs:
          d.wait()

      sdescs = [pltpu.async_copy(rows.at[pl.ds(j * LANES, LANES)],
                                 acc.at[dstv.at[j]], ssem, add=True)
                for j in range(K)]
      for d in sdescs:
        d.wait()
      return carry
    lax.fori_loop(0, ITERS, chunk, 0)

    plsc.subcore_barrier()

    slc = pl.ds(s * ACC_PER_TILE, ACC_PER_TILE)

    @pl.when(c == 0)
    def _():
      pltpu.sync_copy(acc.at[slc], na_hbm.at[slc])

    @pl.when(c == 1)
    def _():
      pltpu.sync_copy(acc.at[slc], nb_hbm.at[slc])

  return k(xa, xb, src2, dst2, zeros)


def _combine_mid_body(x_ref, na_ref, nb_ref, wt_ref, b_ref, ya_ref, yb_ref):
  x = x_ref[...]
  n = jnp.concatenate([na_ref[...], nb_ref[...]], axis=1)
  wt = wt_ref[...]
  y = jnp.dot(x, wt[:D, :], preferred_element_type=jnp.float32)
  y = y + jnp.dot(n, wt[D:, :], preferred_element_type=jnp.float32)
  y = jnp.maximum(y + b_ref[...], 0.0)
  ya_ref[...] = y[:, :H]
  yb_ref[...] = y[:, H:]


def _combine_mid(x, na, nb, wt, b):
  return pl.pallas_call(
      _combine_mid_body,
      grid=(GRID,),
      in_specs=[
          pl.BlockSpec((BM, D), lambda i: (i, 0)),
          pl.BlockSpec((BM, H), lambda i: (i, 0)),
          pl.BlockSpec((BM, H), lambda i: (i, 0)),
          pl.BlockSpec((2 * D, D), lambda i: (0, 0)),
          pl.BlockSpec((1, D), lambda i: (0, 0)),
      ],
      out_specs=(
          pl.BlockSpec((BM, H), lambda i: (i, 0)),
          pl.BlockSpec((BM, H), lambda i: (i, 0)),
      ),
      out_shape=(
          jax.ShapeDtypeStruct((N, H), jnp.float32),
          jax.ShapeDtypeStruct((N, H), jnp.float32),
      ),
  )(x, na, nb, wt, b)


def _combine_final_body(xa_ref, xb_ref, na_ref, nb_ref, wt_ref, b_ref, y_ref):
  x = jnp.concatenate([xa_ref[...], xb_ref[...]], axis=1)
  n = jnp.concatenate([na_ref[...], nb_ref[...]], axis=1)
  wt = wt_ref[...]
  y = jnp.dot(x, wt[:D, :], preferred_element_type=jnp.float32)
  y = y + jnp.dot(n, wt[D:, :], preferred_element_type=jnp.float32)
  y_ref[...] = jnp.maximum(y + b_ref[...], 0.0)


def _combine_final(xa, xb, na, nb, wt, b):
  return pl.pallas_call(
      _combine_final_body,
      grid=(GRID,),
      in_specs=[
          pl.BlockSpec((BM, H), lambda i: (i, 0)),
          pl.BlockSpec((BM, H), lambda i: (i, 0)),
          pl.BlockSpec((BM, H), lambda i: (i, 0)),
          pl.BlockSpec((BM, H), lambda i: (i, 0)),
          pl.BlockSpec((2 * D, D), lambda i: (0, 0)),
          pl.BlockSpec((1, D), lambda i: (0, 0)),
      ],
      out_specs=pl.BlockSpec((BM, D), lambda i: (i, 0)),
      out_shape=jax.ShapeDtypeStruct((N, D), jnp.float32),
  )(xa, xb, na, nb, wt, b)


def kernel(edge_index, emb, W1, b1, W2, b2):
  dst = edge_index[0]
  src = edge_index[1]
  pad = EP - E
  srcp = jnp.concatenate([src, jnp.zeros((pad,), jnp.int32)]).reshape(ROWS, LANES)
  dstp = jnp.concatenate([dst, jnp.full((pad,), N, jnp.int32)]).reshape(ROWS, LANES)
  zeros = jnp.zeros((NACC, H), jnp.float32)

  na, nb = _sc_segment_sum(emb[:, :H], emb[:, H:], srcp, dstp, zeros)
  ya, yb = _combine_mid(emb, na, nb, W1.T, b1.reshape(1, D))

  ma, mb = _sc_segment_sum(ya, yb, srcp, dstp, zeros)
  return _combine_final(ya, yb, ma, mb, W2.T, b2.reshape(1, D))


# speedup vs baseline: 12.5022x; 1.1590x over previous
import functools

import jax
import jax.numpy as jnp
from jax import lax
from jax.experimental import pallas as pl
from jax.experimental.pallas import tpu as pltpu
from jax.experimental.pallas import tpu_sc as plsc

N = 100000
D = 32
H = D // 2
E = 1600000

LANES = 128
K = 4
NTILES = 16
ROWS = 12544
EP = ROWS * LANES
ROWS_PER_TILE = ROWS // NTILES
ITERS = ROWS_PER_TILE // K
NG = ITERS // 4
NACC = 100096
ACC_PER_TILE = NACC // NTILES

BM = 1024
GRID = (N + BM - 1) // BM


def _sc_segment_sum(xa, xb, idxpk, zeros):
  mesh = plsc.VectorSubcoreMesh(core_axis_name="c", subcore_axis_name="s")

  @functools.partial(
      pl.kernel,
      out_type=(
          jax.ShapeDtypeStruct((NACC, H), jnp.float32),
          jax.ShapeDtypeStruct((NACC, H), jnp.float32),
      ),
      mesh=mesh,
      scratch_types=[
          pltpu.VMEM((2 * K, LANES), jnp.int32),
          pltpu.VMEM((2 * K, LANES), jnp.int32),
          pltpu.VMEM((2 * K, LANES), jnp.int32),
          pltpu.VMEM((2 * K, LANES), jnp.int32),
          pltpu.VMEM((K * LANES, H), jnp.float32),
          pltpu.VMEM((K * LANES, H), jnp.float32),
          pltpu.VMEM_SHARED((NACC, H), jnp.float32),
          pltpu.SemaphoreType.DMA,
          pltpu.SemaphoreType.DMA,
          pltpu.SemaphoreType.DMA,
          pltpu.SemaphoreType.DMA,
      ],
      compiler_params=pltpu.CompilerParams(use_tc_tiling_on_sc=False),
  )
  def k(xa_hbm, xb_hbm, idx_hbm, z_hbm, na_hbm, nb_hbm,
        idxv0, idxv1, idxv2, idxv3, rows0, rows1, acc,
        gsem, ssem0, ssem1, isem):
    idxv = (idxv0, idxv1, idxv2, idxv3)
    rows = (rows0, rows1)
    ssem = (ssem0, ssem1)
    c = lax.axis_index("c")
    s = lax.axis_index("s")

    zslc = pl.ds(s * ACC_PER_TILE, ACC_PER_TILE)
    pltpu.sync_copy(z_hbm.at[zslc], acc.at[zslc])
    plsc.subcore_barrier()

    base = s * ITERS

    def drain_scatters(rb):
      for _ in range(K):
        pltpu.make_async_copy(z_hbm.at[pl.ds(0, LANES)],
                              rows[rb].at[pl.ds(0, LANES)], ssem[rb]).wait()

    for b in (0, 1):
      pltpu.async_copy(idx_hbm.at[pl.ds((base + b) * 2 * K, 2 * K)],
                       idxv[b], isem)

    def group(g, carry):
      for b in range(4):
        t = g * 4 + b
        rb = b % 2
        if b < 2:
          pl.when(g >= 1)(lambda rb=rb: drain_scatters(rb))
        else:
          drain_scatters(rb)
        def pf(t=t, b=b):
          pltpu.async_copy(
              idx_hbm.at[pl.ds((base + t + 2) * 2 * K, 2 * K)],
              idxv[(b + 2) % 4], isem)
        if b < 2:
          pf()
        else:
          pl.when(g < NG - 1)(pf)
        pltpu.make_async_copy(idx_hbm.at[pl.ds(0, 2 * K)],
                              idxv[b], isem).wait()
        @pl.when(c == 0)
        def _(b=b, rb=rb):
          descs = [pltpu.async_copy(xa_hbm.at[idxv[b].at[j]],
                                    rows[rb].at[pl.ds(j * LANES, LANES)],
                                    gsem)
                   for j in range(K)]
          for d in descs:
            d.wait()

        @pl.when(c == 1)
        def _(b=b, rb=rb):
          descs = [pltpu.async_copy(xb_hbm.at[idxv[b].at[j]],
                                    rows[rb].at[pl.ds(j * LANES, LANES)],
                                    gsem)
                   for j in range(K)]
          for d in descs:
            d.wait()

        for j in range(K):
          pltpu.async_copy(rows[rb].at[pl.ds(j * LANES, LANES)],
                           acc.at[idxv[b].at[K + j]], ssem[rb], add=True)
      return carry
    lax.fori_loop(0, NG, group, 0)

    for rb in (0, 1):
      drain_scatters(rb)
    plsc.subcore_barrier()

    slc = pl.ds(s * ACC_PER_TILE, ACC_PER_TILE)

    @pl.when(c == 0)
    def _():
      pltpu.sync_copy(acc.at[slc], na_hbm.at[slc])

    @pl.when(c == 1)
    def _():
      pltpu.sync_copy(acc.at[slc], nb_hbm.at[slc])

  return k(xa, xb, idxpk, zeros)


def _combine_mid_body(x_ref, na_ref, nb_ref, wt_ref, b_ref, ya_ref, yb_ref):
  x = x_ref[...]
  n = jnp.concatenate([na_ref[...], nb_ref[...]], axis=1)
  wt = wt_ref[...]
  y = jnp.dot(x, wt[:D, :], preferred_element_type=jnp.float32)
  y = y + jnp.dot(n, wt[D:, :], preferred_element_type=jnp.float32)
  y = jnp.maximum(y + b_ref[...], 0.0)
  ya_ref[...] = y[:, :H]
  yb_ref[...] = y[:, H:]


def _combine_mid(x, na, nb, wt, b):
  return pl.pallas_call(
      _combine_mid_body,
      grid=(GRID,),
      in_specs=[
          pl.BlockSpec((BM, D), lambda i: (i, 0)),
          pl.BlockSpec((BM, H), lambda i: (i, 0)),
          pl.BlockSpec((BM, H), lambda i: (i, 0)),
          pl.BlockSpec((2 * D, D), lambda i: (0, 0)),
          pl.BlockSpec((1, D), lambda i: (0, 0)),
      ],
      out_specs=(
          pl.BlockSpec((BM, H), lambda i: (i, 0)),
          pl.BlockSpec((BM, H), lambda i: (i, 0)),
      ),
      out_shape=(
          jax.ShapeDtypeStruct((N, H), jnp.float32),
          jax.ShapeDtypeStruct((N, H), jnp.float32),
      ),
  )(x, na, nb, wt, b)


def _combine_final_body(xa_ref, xb_ref, na_ref, nb_ref, wt_ref, b_ref, y_ref):
  x = jnp.concatenate([xa_ref[...], xb_ref[...]], axis=1)
  n = jnp.concatenate([na_ref[...], nb_ref[...]], axis=1)
  wt = wt_ref[...]
  y = jnp.dot(x, wt[:D, :], preferred_element_type=jnp.float32)
  y = y + jnp.dot(n, wt[D:, :], preferred_element_type=jnp.float32)
  y_ref[...] = jnp.maximum(y + b_ref[...], 0.0)


def _combine_final(xa, xb, na, nb, wt, b):
  return pl.pallas_call(
      _combine_final_body,
      grid=(GRID,),
      in_specs=[
          pl.BlockSpec((BM, H), lambda i: (i, 0)),
          pl.BlockSpec((BM, H), lambda i: (i, 0)),
          pl.BlockSpec((BM, H), lambda i: (i, 0)),
          pl.BlockSpec((BM, H), lambda i: (i, 0)),
          pl.BlockSpec((2 * D, D), lambda i: (0, 0)),
          pl.BlockSpec((1, D), lambda i: (0, 0)),
      ],
      out_specs=pl.BlockSpec((BM, D), lambda i: (i, 0)),
      out_shape=jax.ShapeDtypeStruct((N, D), jnp.float32),
  )(xa, xb, na, nb, wt, b)


def kernel(edge_index, emb, W1, b1, W2, b2):
  dst = edge_index[0]
  src = edge_index[1]
  pad = EP - E
  srcp = jnp.concatenate([src, jnp.zeros((pad,), jnp.int32)]).reshape(-1, K, LANES)
  dstp = jnp.concatenate([dst, jnp.full((pad,), N, jnp.int32)]).reshape(-1, K, LANES)
  idxpk = jnp.concatenate([srcp, dstp], axis=1).reshape(ROWS * 2, LANES)
  zeros = jnp.zeros((NACC, H), jnp.float32)

  na, nb = _sc_segment_sum(emb[:, :H], emb[:, H:], idxpk, zeros)
  ya, yb = _combine_mid(emb, na, nb, W1.T, b1.reshape(1, D))

  ma, mb = _sc_segment_sum(ya, yb, idxpk, zeros)
  return _combine_final(ya, yb, ma, mb, W2.T, b2.reshape(1, D))


# speedup vs baseline: 15.8376x; 1.2668x over previous
import functools

import jax
import jax.numpy as jnp
from jax import lax
from jax.experimental import pallas as pl
from jax.experimental.pallas import tpu as pltpu
from jax.experimental.pallas import tpu_sc as plsc

N = 100000
D = 32
H = D // 2
E = 1600000

LANES = 128
K = 4
NTILES = 16
ROWS = 12544
EP = ROWS * LANES
ROWS_PER_TILE = ROWS // NTILES
ITERS = ROWS_PER_TILE // K
NG = ITERS // 4
NPAD = 100096
ACC_PER_TILE = NPAD // NTILES
PK = NPAD * H // LANES

BM = 3128
GRID = PK // BM


def _sc_segment_sum(xa, xb, srcp, dstp, zeros):
  mesh = plsc.VectorSubcoreMesh(core_axis_name="c", subcore_axis_name="s")

  @functools.partial(
      pl.kernel,
      out_type=(
          jax.ShapeDtypeStruct((NPAD, H), jnp.float32),
          jax.ShapeDtypeStruct((NPAD, H), jnp.float32),
      ),
      mesh=mesh,
      scratch_types=[
          [pltpu.VMEM((K, LANES), jnp.int32) for _ in range(4)],
          [pltpu.VMEM((K, LANES), jnp.int32) for _ in range(4)],
          [pltpu.VMEM((K * LANES, H), jnp.float32) for _ in range(2)],
          pltpu.VMEM_SHARED((NPAD, H), jnp.float32),
          pltpu.SemaphoreType.DMA,
          [pltpu.SemaphoreType.DMA for _ in range(2)],
          pltpu.SemaphoreType.DMA,
      ],
      compiler_params=pltpu.CompilerParams(use_tc_tiling_on_sc=False),
  )
  def k(xa_hbm, xb_hbm, src_hbm, dst_hbm, z_hbm, na_hbm, nb_hbm,
        srcv, dstv, rows, acc, gsem, ssem, isem):
    c = lax.axis_index("c")
    s = lax.axis_index("s")

    zslc = pl.ds(s * ACC_PER_TILE, ACC_PER_TILE)
    pltpu.sync_copy(z_hbm.at[zslc], acc.at[zslc])
    plsc.subcore_barrier()

    base = s * ROWS_PER_TILE

    def drain_scatters(rb):
      for _ in range(K):
        pltpu.make_async_copy(z_hbm.at[pl.ds(0, LANES)],
                              rows[rb].at[pl.ds(0, LANES)], ssem[rb]).wait()

    def fetch_idx(t, b):
      row0 = base + t * K
      pltpu.async_copy(src_hbm.at[pl.ds(row0, K)], srcv[b], isem)
      pltpu.async_copy(dst_hbm.at[pl.ds(row0, K)], dstv[b], isem)

    def wait_idx(b):
      pltpu.make_async_copy(src_hbm.at[pl.ds(0, K)], srcv[b], isem).wait()
      pltpu.make_async_copy(dst_hbm.at[pl.ds(0, K)], dstv[b], isem).wait()

    for b in (0, 1):
      fetch_idx(b, b)

    def group(g, carry):
      for b in range(4):
        t = g * 4 + b
        rb = b % 2
        if b < 2:
          pl.when(g >= 1)(lambda rb=rb: drain_scatters(rb))
        else:
          drain_scatters(rb)
        if b < 2:
          fetch_idx(t + 2, (b + 2) % 4)
        else:
          pl.when(g < NG - 1)(lambda t=t, b=b: fetch_idx(t + 2, (b + 2) % 4))
        wait_idx(b)

        @pl.when(c == 0)
        def _(b=b, rb=rb):
          descs = [pltpu.async_copy(xa_hbm.at[srcv[b].at[j]],
                                    rows[rb].at[pl.ds(j * LANES, LANES)],
                                    gsem)
                   for j in range(K)]
          for d in descs:
            d.wait()

        @pl.when(c == 1)
        def _(b=b, rb=rb):
          descs = [pltpu.async_copy(xb_hbm.at[srcv[b].at[j]],
                                    rows[rb].at[pl.ds(j * LANES, LANES)],
                                    gsem)
                   for j in range(K)]
          for d in descs:
            d.wait()

        for j in range(K):
          pltpu.async_copy(rows[rb].at[pl.ds(j * LANES, LANES)],
                           acc.at[dstv[b].at[j]], ssem[rb], add=True)
      return carry
    lax.fori_loop(0, NG, group, 0)

    for rb in (0, 1):
      drain_scatters(rb)
    plsc.subcore_barrier()

    slc = pl.ds(s * ACC_PER_TILE, ACC_PER_TILE)

    @pl.when(c == 0)
    def _():
      pltpu.sync_copy(acc.at[slc], na_hbm.at[slc])

    @pl.when(c == 1)
    def _():
      pltpu.sync_copy(acc.at[slc], nb_hbm.at[slc])

  return k(xa, xb, srcp, dstp, zeros)


def _combine_body(xa_ref, xb_ref, na_ref, nb_ref, w_ref, b_ref,
                  ya_ref, yb_ref):
  w = w_ref[...]
  y = jnp.dot(xa_ref[...], w[:128, :], preferred_element_type=jnp.float32)
  y = y + jnp.dot(xb_ref[...], w[128:256, :],
                  preferred_element_type=jnp.float32)
  y = y + jnp.dot(na_ref[...], w[256:384, :],
                  preferred_element_type=jnp.float32)
  y = y + jnp.dot(nb_ref[...], w[384:, :],
                  preferred_element_type=jnp.float32)
  y = jnp.maximum(y + b_ref[...], 0.0)
  ya_ref[...] = y[:, :128]
  yb_ref[...] = y[:, 128:]


def _combine(xa_p, xb_p, na_p, nb_p, wbd, bt):
  return pl.pallas_call(
      _combine_body,
      grid=(GRID,),
      in_specs=[
          pl.BlockSpec((BM, LANES), lambda i: (i, 0)),
          pl.BlockSpec((BM, LANES), lambda i: (i, 0)),
          pl.BlockSpec((BM, LANES), lambda i: (i, 0)),
          pl.BlockSpec((BM, LANES), lambda i: (i, 0)),
          pl.BlockSpec((512, 256), lambda i: (0, 0)),
          pl.BlockSpec((1, 256), lambda i: (0, 0)),
      ],
      out_specs=(
          pl.BlockSpec((BM, LANES), lambda i: (i, 0)),
          pl.BlockSpec((BM, LANES), lambda i: (i, 0)),
      ),
      out_shape=(
          jax.ShapeDtypeStruct((PK, LANES), jnp.float32),
          jax.ShapeDtypeStruct((PK, LANES), jnp.float32),
      ),
  )(xa_p, xb_p, na_p, nb_p, wbd, bt)


def _block_diag_weights(W, b):
  wt = W.T
  base = wt.reshape(4, H, 2, H)
  eye8 = jnp.eye(8, dtype=W.dtype)
  wbd = jnp.einsum("gfho,ij->gifhjo", base, eye8).reshape(512, 256)
  bt = jnp.broadcast_to(b.reshape(2, 1, H), (2, 8, H)).reshape(1, 256)
  return wbd, bt


def kernel(edge_index, emb, W1, b1, W2, b2):
  dst = edge_index[0]
  src = edge_index[1]
  pad = EP - E
  srcp = jnp.concatenate([src, jnp.zeros((pad,), jnp.int32)]).reshape(ROWS, LANES)
  dstp = jnp.concatenate([dst, jnp.full((pad,), N, jnp.int32)]).reshape(ROWS, LANES)
  zeros = jnp.zeros((NPAD, H), jnp.float32)
  xa = jnp.pad(emb[:, :H], ((0, NPAD - N), (0, 0)))
  xb = jnp.pad(emb[:, H:], ((0, NPAD - N), (0, 0)))
  wbd1, bt1 = _block_diag_weights(W1, b1)
  wbd2, bt2 = _block_diag_weights(W2, b2)

  def pk(v):
    return v.reshape(PK, LANES)

  def unpk(v):
    return v.reshape(NPAD, H)

  na, nb = _sc_segment_sum(xa, xb, srcp, dstp, zeros)
  ya_p, yb_p = _combine(pk(xa), pk(xb), pk(na), pk(nb), wbd1, bt1)

  ma, mb = _sc_segment_sum(unpk(ya_p), unpk(yb_p), srcp, dstp, zeros)
  za_p, zb_p = _combine(ya_p, yb_p, pk(ma), pk(mb), wbd2, bt2)

  return jnp.concatenate([unpk(za_p)[:N], unpk(zb_p)[:N]], axis=1)


# speedup vs baseline: 18.2489x; 1.1522x over previous
import functools

import jax
import jax.numpy as jnp
from jax import lax
from jax.experimental import pallas as pl
from jax.experimental.pallas import tpu as pltpu
from jax.experimental.pallas import tpu_sc as plsc

N = 100000
D = 32
H = D // 2
E = 1600000

LANES = 128
K = 4
NTILES = 16
ROWS = 12544
EP = ROWS * LANES
ROWS_PER_TILE = ROWS // NTILES
ITERS = ROWS_PER_TILE // K
NG = ITERS // 4
NPAD = 100096
ACC_PER_TILE = NPAD // NTILES
PK = NPAD * H // LANES

BM = 3128
GRID = PK // BM


def _sc_segment_sum(xs, src_all, dstp, zeros):
  mesh = plsc.VectorSubcoreMesh(core_axis_name="c", subcore_axis_name="s")

  @functools.partial(
      pl.kernel,
      out_type=jax.ShapeDtypeStruct((2 * NPAD, H), jnp.float32),
      mesh=mesh,
      scratch_types=[
          [pltpu.VMEM((K, LANES), jnp.int32) for _ in range(4)],
          [pltpu.VMEM((K, LANES), jnp.int32) for _ in range(4)],
          [pltpu.VMEM((K * LANES, H), jnp.float32) for _ in range(2)],
          pltpu.VMEM_SHARED((NPAD, H), jnp.float32),
          pltpu.SemaphoreType.DMA,
          [pltpu.SemaphoreType.DMA for _ in range(2)],
          pltpu.SemaphoreType.DMA,
      ],
      compiler_params=pltpu.CompilerParams(use_tc_tiling_on_sc=False),
  )
  def k(x_hbm, src_hbm, dst_hbm, z_hbm, nab_hbm,
        srcv, dstv, rows, acc, gsem, ssem, isem):
    c = lax.axis_index("c")
    s = lax.axis_index("s")

    zslc = pl.ds(s * ACC_PER_TILE, ACC_PER_TILE)
    pltpu.sync_copy(z_hbm.at[zslc], acc.at[zslc])
    plsc.subcore_barrier()

    sbase = c * ROWS + s * ROWS_PER_TILE
    dbase = s * ROWS_PER_TILE

    def drain_scatters(rb):
      for _ in range(K):
        pltpu.make_async_copy(z_hbm.at[pl.ds(0, LANES)],
                              rows[rb].at[pl.ds(0, LANES)], ssem[rb]).wait()

    def fetch_idx(t, b):
      pltpu.async_copy(src_hbm.at[pl.ds(sbase + t * K, K)], srcv[b], isem)
      pltpu.async_copy(dst_hbm.at[pl.ds(dbase + t * K, K)], dstv[b], isem)

    def wait_idx(b):
      pltpu.make_async_copy(src_hbm.at[pl.ds(0, K)], srcv[b], isem).wait()
      pltpu.make_async_copy(dst_hbm.at[pl.ds(0, K)], dstv[b], isem).wait()

    for b in (0, 1):
      fetch_idx(b, b)

    def group(g, carry):
      for b in range(4):
        t = g * 4 + b
        rb = b % 2
        if b < 2:
          pl.when(g >= 1)(lambda rb=rb: drain_scatters(rb))
        else:
          drain_scatters(rb)
        if b < 2:
          fetch_idx(t + 2, (b + 2) % 4)
        else:
          pl.when(g < NG - 1)(lambda t=t, b=b: fetch_idx(t + 2, (b + 2) % 4))
        wait_idx(b)
        descs = [pltpu.async_copy(x_hbm.at[srcv[b].at[j]],
                                  rows[rb].at[pl.ds(j * LANES, LANES)], gsem)
                 for j in range(K)]
        for d in descs:
          d.wait()
        for j in range(K):
          pltpu.async_copy(rows[rb].at[pl.ds(j * LANES, LANES)],
                           acc.at[dstv[b].at[j]], ssem[rb], add=True)
      return carry
    lax.fori_loop(0, NG, group, 0)

    for rb in (0, 1):
      drain_scatters(rb)
    plsc.subcore_barrier()

    pltpu.sync_copy(
        acc.at[pl.ds(s * ACC_PER_TILE, ACC_PER_TILE)],
        nab_hbm.at[pl.ds(c * NPAD + s * ACC_PER_TILE, ACC_PER_TILE)])

  return k(xs, src_all, dstp, zeros)


def _combine_mid_body(xa_ref, xb_ref, na_ref, nb_ref, w_ref, b_ref, y_ref):
  w = w_ref[0]
  y = jnp.dot(xa_ref[...], w[:128, :], preferred_element_type=jnp.float32)
  y = y + jnp.dot(xb_ref[...], w[128:256, :],
                  preferred_element_type=jnp.float32)
  y = y + jnp.dot(na_ref[...], w[256:384, :],
                  preferred_element_type=jnp.float32)
  y = y + jnp.dot(nb_ref[...], w[384:, :],
                  preferred_element_type=jnp.float32)
  y_ref[...] = jnp.maximum(y + b_ref[0], 0.0)


def _combine_mid(xs_pk, nab_pk, wbd, bt):
  half = pl.BlockSpec((BM, LANES), lambda i, h: (i, 0))
  other = pl.BlockSpec((BM, LANES), lambda i, h: (i + GRID, 0))
  return pl.pallas_call(
      _combine_mid_body,
      grid=(GRID, 2),
      in_specs=[
          half, other, half, other,
          pl.BlockSpec((1, 512, LANES), lambda i, h: (h, 0, 0)),
          pl.BlockSpec((1, 1, LANES), lambda i, h: (h, 0, 0)),
      ],
      out_specs=pl.BlockSpec((BM, LANES), lambda i, h: (h * GRID + i, 0)),
      out_shape=jax.ShapeDtypeStruct((2 * PK, LANES), jnp.float32),
  )(xs_pk, xs_pk, nab_pk, nab_pk, wbd, bt)


def _combine_final_body(xa_ref, xb_ref, na_ref, nb_ref, w_ref, b_ref, y_ref):
  w = w_ref[...]
  y = jnp.dot(xa_ref[...], w[:128, :], preferred_element_type=jnp.float32)
  y = y + jnp.dot(xb_ref[...], w[128:256, :],
                  preferred_element_type=jnp.float32)
  y = y + jnp.dot(na_ref[...], w[256:384, :],
                  preferred_element_type=jnp.float32)
  y = y + jnp.dot(nb_ref[...], w[384:, :],
                  preferred_element_type=jnp.float32)
  y_ref[...] = jnp.maximum(y + b_ref[...], 0.0)


def _combine_final(xs_pk, nab_pk, wfin, bfin):
  half = pl.BlockSpec((BM, LANES), lambda i: (i, 0))
  other = pl.BlockSpec((BM, LANES), lambda i: (i + GRID, 0))
  return pl.pallas_call(
      _combine_final_body,
      grid=(GRID,),
      in_specs=[
          half, other, half, other,
          pl.BlockSpec((512, 256), lambda i: (0, 0)),
          pl.BlockSpec((1, 256), lambda i: (0, 0)),
      ],
      out_specs=pl.BlockSpec((BM, 256), lambda i: (i, 0)),
      out_shape=jax.ShapeDtypeStruct((PK, 256), jnp.float32),
  )(xs_pk, xs_pk, nab_pk, nab_pk, wfin, bfin)


def _mid_weights(W, b):
  base = W.T.reshape(4, H, 2, H)
  eye8 = jnp.eye(8, dtype=W.dtype)
  wbd = jnp.einsum("gfho,ij->hgifjo", base, eye8).reshape(2, 512, LANES)
  bt = jnp.broadcast_to(b.reshape(2, 1, H), (2, 8, H)).reshape(2, 1, LANES)
  return wbd, bt


def _final_weights(W, b):
  base = W.T.reshape(4, H, 2, H)
  eye8 = jnp.eye(8, dtype=W.dtype)
  wfin = jnp.einsum("gfho,ij->gifjho", base, eye8).reshape(512, 256)
  bfin = jnp.broadcast_to(b.reshape(1, 2, H), (8, 2, H)).reshape(1, 256)
  return wfin, bfin


def kernel(edge_index, emb, W1, b1, W2, b2):
  dst = edge_index[0]
  src = edge_index[1]
  pad = EP - E
  srcp = jnp.concatenate([src, jnp.zeros((pad,), jnp.int32)]).reshape(ROWS, LANES)
  dstp = jnp.concatenate([dst, jnp.full((pad,), N, jnp.int32)]).reshape(ROWS, LANES)
  src_all = jnp.concatenate([srcp, srcp + NPAD])
  zeros = jnp.zeros((NPAD, H), jnp.float32)
  xs = jnp.concatenate([jnp.pad(emb[:, :H], ((0, NPAD - N), (0, 0))),
                        jnp.pad(emb[:, H:], ((0, NPAD - N), (0, 0)))])
  wbd1, bt1 = _mid_weights(W1, b1)
  wfin, bfin = _final_weights(W2, b2)

  def pk(v):
    return v.reshape(-1, LANES)

  nab = _sc_segment_sum(xs, src_all, dstp, zeros)
  yab = _combine_mid(pk(xs), pk(nab), wbd1, bt1)

  mab = _sc_segment_sum(yab.reshape(2 * NPAD, H), src_all, dstp, zeros)
  yfin = _combine_final(yab, pk(mab), wfin, bfin)

  return yfin.reshape(NPAD, D)[:N]
